# baseline (speedup 1.0000x reference)
import functools

import jax
import jax.numpy as jnp
from jax import lax
from jax.experimental import pallas as pl
from jax.experimental.pallas import tpu as pltpu
from jax.experimental.pallas import tpu_sc as plsc

B = 16384
F = 26
CARD = 100000
D = 16
R = F * CARD

NC = 2
NS = 16
NW = NC * NS
L = 16

B_PER_W = B // NW
CHUNK = 128
NCHUNK = B_PER_W // CHUNK
GROUPS = CHUNK // L


def _fm_body(x_ref, emb_ref, lin_ref, bias_ref, out_ref,
             idx_v, emb_g, lin_v, outbuf, bias_v,
             sem_x, sem_emb, sem_lin):
    wid = lax.axis_index("s") * NC + lax.axis_index("c")
    base = wid * B_PER_W

    pltpu.sync_copy(bias_ref, bias_v)

    def chunk_body(c, carry):
        cbase = base + c * CHUNK
        x_cps = [pltpu.async_copy(x_ref.at[pl.ds(f * B + cbase, CHUNK)],
                                  idx_v.at[f], sem_x) for f in range(F)]
        for cp in x_cps:
            cp.wait()

        for f in range(1, F):
            for j in range(GROUPS):
                sl = pl.ds(j * L, L)
                idx_v[f, sl] = idx_v[f, sl] + f * CARD

        emb_cps = []
        lin_cps = []
        for f in range(F):
            for d in range(D):
                emb_cps.append(pltpu.async_copy(
                    emb_ref.at[d].at[idx_v.at[f]],
                    emb_g.at[pl.ds((f * D + d) * CHUNK, CHUNK)], sem_emb))
            lin_cps.append(pltpu.async_copy(
                lin_ref.at[idx_v.at[f]],
                lin_v.at[pl.ds(f * CHUNK, CHUNK)], sem_lin))
        for cp in emb_cps:
            cp.wait()
        for cp in lin_cps:
            cp.wait()

        bias_vec = bias_v[...]

        def group_body(g, gcarry):
            boff = g * L
            s = [jnp.zeros((L,), jnp.float32) for _ in range(D)]
            q = [jnp.zeros((L,), jnp.float32) for _ in range(D)]
            lacc = jnp.zeros((L,), jnp.float32)
            for f in range(F):
                for d in range(D):
                    v = emb_g[pl.ds((f * D + d) * CHUNK + boff, L)]
                    s[d] = s[d] + v
                    q[d] = q[d] + v * v
                lacc = lacc + lin_v[pl.ds(f * CHUNK + boff, L)]
            inter = jnp.zeros((L,), jnp.float32)
            for d in range(D):
                inter = inter + (s[d] * s[d] - q[d])
            outbuf[pl.ds(boff, L)] = lacc + bias_vec + 0.5 * inter
            return gcarry

        lax.fori_loop(0, GROUPS, group_body, 0)
        pltpu.sync_copy(outbuf, out_ref.at[pl.ds(cbase, CHUNK)])
        return carry

    lax.fori_loop(0, NCHUNK, chunk_body, 0)


@jax.jit
def _fm(x, emb_table, lin2, lin_b):
    mesh = plsc.VectorSubcoreMesh(core_axis_name="c", subcore_axis_name="s")
    assert emb_table.shape == (D, R)
    return pl.kernel(
        _fm_body,
        out_type=jax.ShapeDtypeStruct((B,), jnp.float32),
        mesh=mesh,
        compiler_params=pltpu.CompilerParams(
            needs_layout_passes=False, use_tc_tiling_on_sc=False),
        scratch_types=[
            pltpu.VMEM((F, CHUNK), jnp.int32),
            pltpu.VMEM((F * D * CHUNK,), jnp.float32),
            pltpu.VMEM((F * CHUNK,), jnp.float32),
            pltpu.VMEM((CHUNK,), jnp.float32),
            pltpu.VMEM((L,), jnp.float32),
            pltpu.SemaphoreType.DMA,
            pltpu.SemaphoreType.DMA,
            pltpu.SemaphoreType.DMA,
        ],
    )(x, emb_table, lin2, lin_b)


def kernel(x, emb_table, lin_w, lin_b):
    bias16 = jnp.broadcast_to(lin_b, (L,))
    xf = x.T.reshape(F * B)
    emb2 = jax.lax.optimization_barrier(emb_table.T)
    out = _fm(xf, emb2, lin_w, bias16)
    return out.reshape(B, 1)


# speedup vs baseline: 3.0249x; 3.0249x over previous
import functools

import jax
import jax.numpy as jnp
from jax import lax
from jax.experimental import pallas as pl
from jax.experimental.pallas import tpu as pltpu
from jax.experimental.pallas import tpu_sc as plsc

B = 16384
F = 26
CARD = 100000
D = 16
R = F * CARD

NC = 2
NS = 16
NW = NC * NS
L = 16
PACK = 8
W = D * PACK

NRB = R // W
TAIL = R - NRB * W
RB_PER_W = -(-NRB // NW)

B_PER_W = B // NW
CHUNK = 32
NCHUNK = B_PER_W // CHUNK
GROUPS = CHUNK // L


def _transpose_block(inbuf, tbuf, lane, ncols):
    for mm in range(ncols // PACK):
        for k in range(PACK):
            col = jnp.full((L,), mm * PACK + k, jnp.int32)
            v = plsc.load_gather(inbuf, [lane, col])
            tbuf[mm, pl.ds(k * L, L)] = v


def _relayout_body(emb_ref, tail_ref, out_ref, in0, in1, t0, t1,
                   sem_in, sem_out):
    wid = lax.axis_index("s") * NC + lax.axis_index("c")
    lane = lax.iota(jnp.int32, L)
    base = wid * RB_PER_W
    nmine = jnp.minimum(RB_PER_W, jnp.maximum(NRB - base, 0))

    def fetch(rb, buf):
        safe = jnp.clip(rb, 0, NRB - 1)
        return pltpu.async_copy(
            emb_ref.at[:, pl.ds(safe * W, W)], buf, sem_in)

    def drain_in():
        pltpu.make_async_copy(
            emb_ref.at[:, pl.ds(0, W)], in0, sem_in).wait()

    def drain_out():
        pltpu.make_async_copy(
            t0, out_ref.at[pl.ds(0, L), :], sem_out).wait()

    @pl.when(nmine > 0)
    def _():
        fetch(base, in0)

    def loop_body(i, carry):
        @pl.when(jnp.logical_and(i + 1 < nmine, i % 2 == 0))
        def _():
            fetch(base + i + 1, in1)

        @pl.when(jnp.logical_and(i + 1 < nmine, i % 2 == 1))
        def _():
            fetch(base + i + 1, in0)

        drain_in()

        @pl.when(i >= 2)
        def _():
            drain_out()

        @pl.when(i % 2 == 0)
        def _():
            _transpose_block(in0, t0, lane, W)
            pltpu.async_copy(
                t0, out_ref.at[pl.ds((base + i) * L, L), :], sem_out)

        @pl.when(i % 2 == 1)
        def _():
            _transpose_block(in1, t1, lane, W)
            pltpu.async_copy(
                t1, out_ref.at[pl.ds((base + i) * L, L), :], sem_out)
        return carry

    lax.fori_loop(0, nmine, loop_body, 0)

    @pl.when(nmine >= 2)
    def _():
        drain_out()

    @pl.when(nmine == 1)
    def _():
        drain_out()

    @pl.when(nmine >= 2)
    def _():
        drain_out()

    @pl.when(wid == 0)
    def _():
        pltpu.async_copy(
            tail_ref, out_ref.at[pl.ds(NRB * L, TAIL // PACK), :],
            sem_out).wait()


def _fm_body(x_ref, emb_ref, lin_ref, bias_ref, out_ref,
             xbuf, idx_v, flat_v, rows_v, lin_v, outbuf, bias_v,
             sem_x, sem_emb, sem_lin):
    wid = lax.axis_index("s") * NC + lax.axis_index("c")
    base = wid * B_PER_W

    pltpu.sync_copy(bias_ref, bias_v)
    lane = lax.iota(jnp.int32, L)

    def chunk_body(c, carry):
        cbase = base + c * CHUNK
        pltpu.async_copy(x_ref.at[pl.ds(cbase * F, CHUNK * F)], xbuf,
                         sem_x).wait()

        for f in range(F):
            for j in range(GROUPS):
                bvec = j * L + lane
                raw = plsc.load_gather(xbuf, [bvec * F + f])
                flat = raw + f * CARD
                idx_v[f, pl.ds(j * L, L)] = lax.shift_right_logical(flat, 3)
                flat_v[f, pl.ds(j * L, L)] = flat

        emb_cps = []
        lin_cps = []
        for f in range(F):
            emb_cps.append(pltpu.async_copy(
                emb_ref.at[idx_v.at[f]],
                rows_v.at[pl.ds(f * CHUNK, CHUNK), :], sem_emb))
            lin_cps.append(pltpu.async_copy(
                lin_ref.at[flat_v.at[f]],
                lin_v.at[pl.ds(f * CHUNK, CHUNK)], sem_lin))
        for cp in emb_cps:
            cp.wait()
        for cp in lin_cps:
            cp.wait()

        bias_vec = bias_v[...]

        def group_body(g, gcarry):
            boff = g * L
            bvec = boff + lane
            s = [jnp.zeros((L,), jnp.float32) for _ in range(D)]
            q = [jnp.zeros((L,), jnp.float32) for _ in range(D)]
            lacc = jnp.zeros((L,), jnp.float32)
            for f in range(F):
                ridx = bvec + f * CHUNK
                flat = flat_v[f, pl.ds(boff, L)]
                colb = lax.shift_left(jnp.bitwise_and(flat, 7), 4)
                for d in range(D):
                    v = plsc.load_gather(rows_v, [ridx, colb + d])
                    s[d] = s[d] + v
                    q[d] = q[d] + v * v
                lacc = lacc + plsc.load_gather(lin_v, [ridx])
            inter = jnp.zeros((L,), jnp.float32)
            for d in range(D):
                inter = inter + (s[d] * s[d] - q[d])
            outbuf[pl.ds(boff, L)] = lacc + bias_vec + 0.5 * inter
            return gcarry

        lax.fori_loop(0, GROUPS, group_body, 0)
        pltpu.sync_copy(outbuf, out_ref.at[pl.ds(cbase, CHUNK)])
        return carry

    lax.fori_loop(0, NCHUNK, chunk_body, 0)


@jax.jit
def _fm2(x, emb_t, tailpack, lin2, lin_b):
    mesh = plsc.VectorSubcoreMesh(core_axis_name="c", subcore_axis_name="s")
    params = pltpu.CompilerParams(
        needs_layout_passes=False, use_tc_tiling_on_sc=True)
    emb_packed = pl.kernel(
        _relayout_body,
        out_type=jax.ShapeDtypeStruct((R // PACK, W), jnp.float32),
        mesh=mesh,
        compiler_params=params,
        scratch_types=[
            pltpu.VMEM((D, W), jnp.float32),
            pltpu.VMEM((D, W), jnp.float32),
            pltpu.VMEM((L, W), jnp.float32),
            pltpu.VMEM((L, W), jnp.float32),
            pltpu.SemaphoreType.DMA,
            pltpu.SemaphoreType.DMA,
        ],
    )(emb_t, tailpack)
    return pl.kernel(
        _fm_body,
        out_type=jax.ShapeDtypeStruct((B,), jnp.float32),
        mesh=mesh,
        compiler_params=params,
        scratch_types=[
            pltpu.VMEM((CHUNK * F,), jnp.int32),
            pltpu.VMEM((F, CHUNK), jnp.int32),
            pltpu.VMEM((F, CHUNK), jnp.int32),
            pltpu.VMEM((F * CHUNK, W), jnp.float32),
            pltpu.VMEM((F * CHUNK,), jnp.float32),
            pltpu.VMEM((CHUNK,), jnp.float32),
            pltpu.VMEM((L,), jnp.float32),
            pltpu.SemaphoreType.DMA,
            pltpu.SemaphoreType.DMA,
            pltpu.SemaphoreType.DMA,
        ],
    )(x, emb_packed, lin2, lin_b)


def kernel(x, emb_table, lin_w, lin_b):
    bias16 = jnp.broadcast_to(lin_b, (L,))
    tailpack = emb_table[R - TAIL:, :].reshape(TAIL // PACK, W)
    out = _fm2(x.reshape(B * F), emb_table.T, tailpack, lin_w, bias16)
    return out.reshape(B, 1)


# speedup vs baseline: 3.0266x; 1.0005x over previous
import functools

import jax
import jax.numpy as jnp
from jax import lax
from jax.experimental import pallas as pl
from jax.experimental.pallas import tpu as pltpu
from jax.experimental.pallas import tpu_sc as plsc

B = 16384
F = 26
CARD = 100000
D = 16
R = F * CARD

NC = 2
NS = 16
NW = NC * NS
L = 16
PACK = 8
W = D * PACK

NRB = R // W
TAIL = R - NRB * W
RB_PER_W = -(-NRB // NW)

B_PER_W = B // NW
CHUNK = 32
NCHUNK = B_PER_W // CHUNK
GROUPS = CHUNK // L


def _transpose_block(inbuf, tbuf, lane):
    for mm in range(W // PACK):
        for k in range(PACK):
            col = jnp.full((L,), mm * PACK + k, jnp.int32)
            v = plsc.load_gather(inbuf, [lane, col])
            tbuf[mm, pl.ds(k * L, L)] = v


def _relayout_body(emb_ref, tail_ref, out_ref, in0, in1, t0, t1,
                   sem_in, sem_out):
    wid = lax.axis_index("s") * NC + lax.axis_index("c")
    lane = lax.iota(jnp.int32, L)
    base = wid * RB_PER_W
    nmine = jnp.minimum(RB_PER_W, jnp.maximum(NRB - base, 0))

    def fetch(rb, buf):
        safe = jnp.clip(rb, 0, NRB - 1)
        return pltpu.async_copy(
            emb_ref.at[:, pl.ds(safe * W, W)], buf, sem_in)

    def drain_in():
        pltpu.make_async_copy(
            emb_ref.at[:, pl.ds(0, W)], in0, sem_in).wait()

    def drain_out():
        pltpu.make_async_copy(
            t0, out_ref.at[pl.ds(0, L), :], sem_out).wait()

    @pl.when(nmine > 0)
    def _():
        fetch(base, in0)

    def loop_body(i, carry):
        @pl.when(jnp.logical_and(i + 1 < nmine, i % 2 == 0))
        def _():
            fetch(base + i + 1, in1)

        @pl.when(jnp.logical_and(i + 1 < nmine, i % 2 == 1))
        def _():
            fetch(base + i + 1, in0)

        drain_in()

        @pl.when(i >= 2)
        def _():
            drain_out()

        @pl.when(i % 2 == 0)
        def _():
            _transpose_block(in0, t0, lane)
            pltpu.async_copy(
                t0, out_ref.at[pl.ds((base + i) * L, L), :], sem_out)

        @pl.when(i % 2 == 1)
        def _():
            _transpose_block(in1, t1, lane)
            pltpu.async_copy(
                t1, out_ref.at[pl.ds((base + i) * L, L), :], sem_out)
        return carry

    lax.fori_loop(0, nmine, loop_body, 0)

    @pl.when(nmine >= 2)
    def _():
        drain_out()

    @pl.when(nmine == 1)
    def _():
        drain_out()

    @pl.when(nmine >= 2)
    def _():
        drain_out()

    @pl.when(wid == 0)
    def _():
        pltpu.async_copy(
            tail_ref, out_ref.at[pl.ds(NRB * L, TAIL // PACK), :],
            sem_out).wait()


def _fm_body(x_ref, emb_ref, lin_ref, bias_ref, out_ref,
             xbuf, idx_v, flat_v, rows_v, lin_v, outbuf, bias_v,
             sem_x, sem_emb, sem_lin):
    wid = lax.axis_index("s") * NC + lax.axis_index("c")
    base = wid * B_PER_W

    pltpu.sync_copy(bias_ref, bias_v)
    lane = lax.iota(jnp.int32, L)

    def chunk_body(c, carry):
        cbase = base + c * CHUNK
        pltpu.async_copy(x_ref.at[pl.ds(cbase * F, CHUNK * F)], xbuf,
                         sem_x).wait()

        for f in range(F):
            for j in range(GROUPS):
                bvec = j * L + lane
                raw = plsc.load_gather(xbuf, [bvec * F + f])
                flat = raw + f * CARD
                idx_v[f, pl.ds(j * L, L)] = lax.shift_right_logical(flat, 3)
                flat_v[f, pl.ds(j * L, L)] = flat

        emb_cps = []
        lin_cps = []
        for f in range(F):
            emb_cps.append(pltpu.async_copy(
                emb_ref.at[idx_v.at[f]],
                rows_v.at[pl.ds(f * CHUNK, CHUNK), :], sem_emb))
            lin_cps.append(pltpu.async_copy(
                lin_ref.at[flat_v.at[f]],
                lin_v.at[pl.ds(f * CHUNK, CHUNK)], sem_lin))
        for cp in emb_cps:
            cp.wait()
        for cp in lin_cps:
            cp.wait()

        bias_vec = bias_v[...]

        def group_body(g, gcarry):
            boff = g * L
            bvec = boff + lane
            s = [jnp.zeros((L,), jnp.float32) for _ in range(D)]
            q = [jnp.zeros((L,), jnp.float32) for _ in range(D)]
            lacc = jnp.zeros((L,), jnp.float32)
            for f in range(F):
                ridx = bvec + f * CHUNK
                flat = flat_v[f, pl.ds(boff, L)]
                colb = lax.shift_left(jnp.bitwise_and(flat, 7), 4)
                for d in range(D):
                    v = plsc.load_gather(rows_v, [ridx, colb + d])
                    s[d] = s[d] + v
                    q[d] = q[d] + v * v
                lacc = lacc + plsc.load_gather(lin_v, [ridx])
            inter = jnp.zeros((L,), jnp.float32)
            for d in range(D):
                inter = inter + (s[d] * s[d] - q[d])
            outbuf[pl.ds(boff, L)] = lacc + bias_vec + 0.5 * inter
            return gcarry

        lax.fori_loop(0, GROUPS, group_body, 0)
        pltpu.sync_copy(outbuf, out_ref.at[pl.ds(cbase, CHUNK)])
        return carry

    lax.fori_loop(0, NCHUNK, chunk_body, 0)


@jax.jit
def _fm2(x, emb_t, tailpack, lin2, lin_b):
    mesh = plsc.VectorSubcoreMesh(core_axis_name="c", subcore_axis_name="s")
    params = pltpu.CompilerParams(
        needs_layout_passes=False, use_tc_tiling_on_sc=True)
    emb_packed = pl.kernel(
        _relayout_body,
        out_type=jax.ShapeDtypeStruct((R // PACK, W), jnp.float32),
        mesh=mesh,
        compiler_params=params,
        scratch_types=[
            pltpu.VMEM((D, W), jnp.float32),
            pltpu.VMEM((D, W), jnp.float32),
            pltpu.VMEM((L, W), jnp.float32),
            pltpu.VMEM((L, W), jnp.float32),
            pltpu.SemaphoreType.DMA,
            pltpu.SemaphoreType.DMA,
        ],
    )(emb_t, tailpack)
    return pl.kernel(
        _fm_body,
        out_type=jax.ShapeDtypeStruct((B,), jnp.float32),
        mesh=mesh,
        compiler_params=params,
        scratch_types=[
            pltpu.VMEM((CHUNK * F,), jnp.int32),
            pltpu.VMEM((F, CHUNK), jnp.int32),
            pltpu.VMEM((F, CHUNK), jnp.int32),
            pltpu.VMEM((F * CHUNK, W), jnp.float32),
            pltpu.VMEM((F * CHUNK,), jnp.float32),
            pltpu.VMEM((CHUNK,), jnp.float32),
            pltpu.VMEM((L,), jnp.float32),
            pltpu.SemaphoreType.DMA,
            pltpu.SemaphoreType.DMA,
            pltpu.SemaphoreType.DMA,
        ],
    )(x, emb_packed, lin2, lin_b)


def kernel(x, emb_table, lin_w, lin_b):
    bias16 = jnp.broadcast_to(lin_b, (L,))
    tailpack = emb_table[R - TAIL:, :].reshape(TAIL // PACK, W)
    out = _fm2(x.reshape(B * F), emb_table.T, tailpack, lin_w, bias16)
    return out.reshape(B, 1)
